# baseline (speedup 1.0000x reference)
import functools

import jax
import jax.numpy as jnp
from jax import lax
from jax.experimental import pallas as pl
from jax.experimental.pallas import tpu as pltpu
from jax.experimental.pallas import tpu_sc as plsc

B, N, D = 64, 2048, 1024
H, QD, QH = 128, 4, 32
K = 64
NEG = -1e9
VALID_THRESH = -1e8
BPS = 1


def _score_pool_body(featsA_ref, featsB_ref, maskf_ref, segq_ref, W1_ref,
                     b1_ref, W2_ref, b2_ref, Q1_ref, qb1_ref, Q2_ref,
                     qb2_ref, weights_ref, attnh_ref, idx_ref, validf_ref,
                     fidx_ref, wk_ref, logits_sc):
    fparts_refs = (featsA_ref, featsB_ref)
    bb = pl.program_id(0)
    for t in range(BPS):
        b = bb * BPS + t
        DQ = D // 2
        fparts = [r[t] for r in fparts_refs]
        h = jnp.tanh(
            jnp.dot(fparts[0], W1_ref[pl.ds(0, DQ), :],
                    preferred_element_type=jnp.float32)
            + jnp.dot(fparts[1], W1_ref[pl.ds(DQ, DQ), :],
                      preferred_element_type=jnp.float32)
            + b1_ref[...])
        ev = lax.dot_general(W2_ref[...], h, (((0,), (1,)), ((), ())),
                             preferred_element_type=jnp.float32)
        q = segq_ref[t]
        qh = jnp.maximum(
            jnp.dot(q, Q1_ref[...], preferred_element_type=jnp.float32)
            + qb1_ref[...], 0.0)
        ql = lax.dot_general(Q2_ref[...], qh, (((0,), (1,)), ((), ())),
                             preferred_element_type=jnp.float32)
        logit = ev + ql + (b2_ref[0, 0] + qb2_ref[0, 0])
        m = maskf_ref[t]
        masked = jnp.where(m > 0.0, logit, NEG)
        logits_sc[pl.ds(b, 1)] = masked
        mx = jnp.max(masked)
        e = jnp.exp(masked - mx)
        s = jnp.sum(e)
        w = (e / s) * m
        tt = jnp.sum(w)
        w = w / jnp.maximum(tt, 1e-8)
        weights_ref[t] = w
        wcol = jnp.reshape(w, (N, 1))
        for j, fp in enumerate(fparts):
            attnh_ref[t, :, pl.ds(j * DQ, DQ)] = (
                0.5 * jnp.sum(fp * wcol, axis=0, keepdims=True))

    @pl.when(bb == B // BPS - 1)
    def _topk_tail():
        cur0 = logits_sc[...]
        iota_n = lax.broadcasted_iota(jnp.int32, (B, N), 1)
        kcol = lax.broadcasted_iota(jnp.int32, (B, K), 1)

        def step(k, carry):
            cur, idxacc, valacc = carry
            mxk = jnp.max(cur, axis=1, keepdims=True)
            cand = jnp.where(cur == mxk, iota_n, N)
            idx = jnp.min(cand, axis=1, keepdims=True)
            sel = kcol == k
            idxacc = jnp.where(sel, idx, idxacc)
            valacc = jnp.where(sel, mxk, valacc)
            cur = jnp.where(iota_n == idx, -jnp.inf, cur)
            return cur, idxacc, valacc

        idxacc0 = jnp.zeros((B, K), jnp.int32)
        valacc0 = jnp.full((B, K), NEG, jnp.float32)
        _, idxacc, valacc = lax.fori_loop(
            0, K, step, (cur0, idxacc0, valacc0))

        validf = (valacc > VALID_THRESH).astype(jnp.float32)
        cnt = jnp.sum(validf, axis=1, keepdims=True)
        scale = 0.5 / jnp.maximum(cnt, 1.0)
        rowoff = lax.broadcasted_iota(jnp.int32, (B, K), 0) * N
        idx_ref[...] = idxacc
        validf_ref[...] = validf
        fidx_ref[...] = idxacc + rowoff
        wk_ref[...] = jnp.broadcast_to(
            (validf * scale)[:, :, None], (B, K, 16))


def _score_pool(feats, maskf, segq, W1, b1, W2, b2, Q1, qb1, Q2, qb2):
    full = lambda shape: pl.BlockSpec(shape, lambda b: (0,) * len(shape))
    return pl.pallas_call(
        _score_pool_body,
        grid=(B // BPS,),
        in_specs=[
            pl.BlockSpec((BPS, N, D // 2), lambda b: (b, 0, 0)),
            pl.BlockSpec((BPS, N, D // 2), lambda b: (b, 0, 1)),
            pl.BlockSpec((BPS, 1, N), lambda b: (b, 0, 0)),
            pl.BlockSpec((BPS, N, QD), lambda b: (b, 0, 0)),
            full((D, H)), full((1, H)), full((H, 1)), full((1, 1)),
            full((QD, QH)), full((1, QH)), full((QH, 1)), full((1, 1)),
        ],
        out_specs=[
            pl.BlockSpec((BPS, 1, N), lambda b: (b, 0, 0)),
            pl.BlockSpec((BPS, 1, D), lambda b: (b, 0, 0)),
            pl.BlockSpec((B, K), lambda b: (0, 0)),
            pl.BlockSpec((B, K), lambda b: (0, 0)),
            pl.BlockSpec((B, K), lambda b: (0, 0)),
            pl.BlockSpec((B, K, 16), lambda b: (0, 0, 0)),
        ],
        out_shape=[
            jax.ShapeDtypeStruct((B, 1, N), jnp.float32),
            jax.ShapeDtypeStruct((B, 1, D), jnp.float32),
            jax.ShapeDtypeStruct((B, K), jnp.int32),
            jax.ShapeDtypeStruct((B, K), jnp.float32),
            jax.ShapeDtypeStruct((B, K), jnp.int32),
            jax.ShapeDtypeStruct((B, K, 16), jnp.float32),
        ],
        scratch_shapes=[pltpu.VMEM((B, N), jnp.float32)],
        compiler_params=pltpu.CompilerParams(
            dimension_semantics=("arbitrary",)),
    )(feats, feats, maskf, segq, W1, b1, W2, b2, Q1, qb1, Q2, qb2)


HR = 32
GW = 16


def _sc_gather_body(feats_hbm, fidx_hbm, wk_hbm, attnh_hbm, out_hbm,
                    idx2_v, rbuf, wk2_v, acc2_v, sem_a, sem_b):
    wid = lax.axis_index("s") * 2 + lax.axis_index("c")
    b0 = wid * 2
    pltpu.sync_copy(fidx_hbm.at[pl.ds(b0, 2)], idx2_v)
    pltpu.sync_copy(wk_hbm.at[pl.ds(b0, 2)], wk2_v)
    pltpu.sync_copy(attnh_hbm.at[pl.ds(b0, 2)], acc2_v)
    sems = (sem_a, sem_b)

    def start(c):
        bi, half = c // 2, c % 2
        return pltpu.async_copy(
            feats_hbm.at[idx2_v.at[bi, pl.ds(half * HR, HR)]],
            rbuf.at[c % 2], sems[c % 2])

    pending = {0: start(0)}
    for c in range(4):
        if c < 3:
            pending[c + 1] = start(c + 1)
        pending[c].wait()
        bi, half = c // 2, c % 2
        koff = half * HR
        for g in range(D // (16 * GW)):
            base = g * 16 * GW

            def kbody(k, accs, bi=bi, koff=koff, base=base, c=c):
                wkv = wk2_v[bi, koff + k, :]
                return tuple(
                    accs[i] + rbuf[c % 2, k, pl.ds(base + i * 16, 16)] * wkv
                    for i in range(GW))

            acc0 = tuple(
                acc2_v[bi, pl.ds(base + i * 16, 16)] for i in range(GW))
            accs = lax.fori_loop(0, HR, kbody, acc0)
            for i in range(GW):
                acc2_v[bi, pl.ds(base + i * 16, 16)] = accs[i]
        if half == 1:
            pltpu.sync_copy(acc2_v.at[bi], out_hbm.at[b0 + bi])


@functools.cache
def _sc_gather_kernel():
    return pl.kernel(
        _sc_gather_body,
        out_type=jax.ShapeDtypeStruct((B, D), jnp.float32),
        mesh=plsc.VectorSubcoreMesh(
            core_axis_name="c", subcore_axis_name="s",
            num_cores=2, num_subcores=16),
        scratch_types=[
            pltpu.VMEM((2, K), jnp.int32),
            pltpu.VMEM((2, HR, D), jnp.float32),
            pltpu.VMEM((2, K, 16), jnp.float32),
            pltpu.VMEM((2, D), jnp.float32),
            pltpu.SemaphoreType.DMA,
            pltpu.SemaphoreType.DMA,
        ],
    )


def kernel(feats, mask, seg_quality, W1, b1, W2, b2, Q1, qb1, Q2, qb2):
    maskf = mask.astype(jnp.float32).reshape(B, 1, N)
    weights, attnh, topk_idx, validf, fidx, wk = _score_pool(
        feats, maskf, seg_quality, W1,
        b1.reshape(1, H), W2, b2.reshape(1, 1),
        Q1, qb1.reshape(1, QH), Q2, qb2.reshape(1, 1))
    weights = weights.reshape(B, N)
    attnh = attnh.reshape(B, D)
    feats2d = feats.reshape(B * N, D)
    pooled = _sc_gather_kernel()(feats2d, fidx, wk, attnh)
    return (pooled, weights, topk_idx, validf.astype(bool))


# speedup vs baseline: 1.1439x; 1.1439x over previous
import functools

import jax
import jax.numpy as jnp
from jax import lax
from jax.experimental import pallas as pl
from jax.experimental.pallas import tpu as pltpu
from jax.experimental.pallas import tpu_sc as plsc

B, N, D = 64, 2048, 1024
H, QD, QH = 128, 4, 32
K = 64
NEG = -1e9
VALID_THRESH = -1e8
BPS = 2


def _score_pool_body(featsA_ref, featsB_ref, maskf_ref, segq_ref, W1_ref,
                     b1_ref, W2_ref, b2_ref, Q1_ref, qb1_ref, Q2_ref,
                     qb2_ref, weights_ref, attnh_ref, idx_ref, validf_ref,
                     fidx_ref, wk_ref, logits_sc):
    fparts_refs = (featsA_ref, featsB_ref)
    bb = pl.program_id(0)
    for t in range(BPS):
        b = bb * BPS + t
        DQ = D // 2
        fparts = [r[t] for r in fparts_refs]
        h = jnp.tanh(
            jnp.dot(fparts[0], W1_ref[pl.ds(0, DQ), :],
                    preferred_element_type=jnp.float32)
            + jnp.dot(fparts[1], W1_ref[pl.ds(DQ, DQ), :],
                      preferred_element_type=jnp.float32)
            + b1_ref[...])
        ev = lax.dot_general(W2_ref[...], h, (((0,), (1,)), ((), ())),
                             preferred_element_type=jnp.float32)
        q = segq_ref[t]
        qh = jnp.maximum(
            jnp.dot(q, Q1_ref[...], preferred_element_type=jnp.float32)
            + qb1_ref[...], 0.0)
        ql = lax.dot_general(Q2_ref[...], qh, (((0,), (1,)), ((), ())),
                             preferred_element_type=jnp.float32)
        logit = ev + ql + (b2_ref[0, 0] + qb2_ref[0, 0])
        m = maskf_ref[t]
        masked = jnp.where(m > 0.0, logit, NEG)
        logits_sc[pl.ds(b, 1)] = masked
        mx = jnp.max(masked)
        e = jnp.exp(masked - mx)
        s = jnp.sum(e)
        w = (e / s) * m
        tt = jnp.sum(w)
        w = w / jnp.maximum(tt, 1e-8)
        weights_ref[t] = w
        wcol = jnp.reshape(w, (N, 1))
        for j, fp in enumerate(fparts):
            attnh_ref[t, :, pl.ds(j * DQ, DQ)] = (
                0.5 * jnp.sum(fp * wcol, axis=0, keepdims=True))

    @pl.when(bb == B // BPS - 1)
    def _topk_tail():
        cur0 = logits_sc[...]
        iota_n = lax.broadcasted_iota(jnp.int32, (B, N), 1)
        kcol = lax.broadcasted_iota(jnp.int32, (B, K), 1)

        def step(k, carry):
            cur, idxacc, valacc = carry
            mxk = jnp.max(cur, axis=1, keepdims=True)
            cand = jnp.where(cur == mxk, iota_n, N)
            idx = jnp.min(cand, axis=1, keepdims=True)
            sel = kcol == k
            idxacc = jnp.where(sel, idx, idxacc)
            valacc = jnp.where(sel, mxk, valacc)
            cur = jnp.where(iota_n == idx, -jnp.inf, cur)
            return cur, idxacc, valacc

        idxacc0 = jnp.zeros((B, K), jnp.int32)
        valacc0 = jnp.full((B, K), NEG, jnp.float32)
        _, idxacc, valacc = lax.fori_loop(
            0, K, step, (cur0, idxacc0, valacc0))

        validf = (valacc > VALID_THRESH).astype(jnp.float32)
        cnt = jnp.sum(validf, axis=1, keepdims=True)
        scale = 0.5 / jnp.maximum(cnt, 1.0)
        rowoff = lax.broadcasted_iota(jnp.int32, (B, K), 0) * N
        idx_ref[...] = idxacc
        validf_ref[...] = validf
        fidx_ref[...] = idxacc + rowoff
        wk_ref[...] = jnp.broadcast_to(
            (validf * scale)[:, :, None], (B, K, 16))


def _score_pool(feats, maskf, segq, W1, b1, W2, b2, Q1, qb1, Q2, qb2):
    full = lambda shape: pl.BlockSpec(shape, lambda b: (0,) * len(shape))
    return pl.pallas_call(
        _score_pool_body,
        grid=(B // BPS,),
        in_specs=[
            pl.BlockSpec((BPS, N, D // 2), lambda b: (b, 0, 0)),
            pl.BlockSpec((BPS, N, D // 2), lambda b: (b, 0, 1)),
            pl.BlockSpec((BPS, 1, N), lambda b: (b, 0, 0)),
            pl.BlockSpec((BPS, N, QD), lambda b: (b, 0, 0)),
            full((D, H)), full((1, H)), full((H, 1)), full((1, 1)),
            full((QD, QH)), full((1, QH)), full((QH, 1)), full((1, 1)),
        ],
        out_specs=[
            pl.BlockSpec((BPS, 1, N), lambda b: (b, 0, 0)),
            pl.BlockSpec((BPS, 1, D), lambda b: (b, 0, 0)),
            pl.BlockSpec((B, K), lambda b: (0, 0)),
            pl.BlockSpec((B, K), lambda b: (0, 0)),
            pl.BlockSpec((B, K), lambda b: (0, 0)),
            pl.BlockSpec((B, K, 16), lambda b: (0, 0, 0)),
        ],
        out_shape=[
            jax.ShapeDtypeStruct((B, 1, N), jnp.float32),
            jax.ShapeDtypeStruct((B, 1, D), jnp.float32),
            jax.ShapeDtypeStruct((B, K), jnp.int32),
            jax.ShapeDtypeStruct((B, K), jnp.float32),
            jax.ShapeDtypeStruct((B, K), jnp.int32),
            jax.ShapeDtypeStruct((B, K, 16), jnp.float32),
        ],
        scratch_shapes=[pltpu.VMEM((B, N), jnp.float32)],
        compiler_params=pltpu.CompilerParams(
            dimension_semantics=("arbitrary",)),
    )(feats, feats, maskf, segq, W1, b1, W2, b2, Q1, qb1, Q2, qb2)


HR = 32
GW = 16


def _sc_gather_body(feats_hbm, fidx_hbm, wk_hbm, attnh_hbm, out_hbm,
                    idx2_v, rbuf, wk2_v, acc2_v, sem_a, sem_b):
    wid = lax.axis_index("s") * 2 + lax.axis_index("c")
    b0 = wid * 2
    pltpu.sync_copy(fidx_hbm.at[pl.ds(b0, 2)], idx2_v)
    pltpu.sync_copy(wk_hbm.at[pl.ds(b0, 2)], wk2_v)
    pltpu.sync_copy(attnh_hbm.at[pl.ds(b0, 2)], acc2_v)
    sems = (sem_a, sem_b)

    def start(c):
        bi, half = c // 2, c % 2
        return pltpu.async_copy(
            feats_hbm.at[idx2_v.at[bi, pl.ds(half * HR, HR)]],
            rbuf.at[c % 2], sems[c % 2])

    pending = {0: start(0)}
    for c in range(4):
        if c < 3:
            pending[c + 1] = start(c + 1)
        pending[c].wait()
        bi, half = c // 2, c % 2
        koff = half * HR
        for g in range(D // (16 * GW)):
            base = g * 16 * GW

            def kbody(k, accs, bi=bi, koff=koff, base=base, c=c):
                wkv = wk2_v[bi, koff + k, :]
                return tuple(
                    accs[i] + rbuf[c % 2, k, pl.ds(base + i * 16, 16)] * wkv
                    for i in range(GW))

            acc0 = tuple(
                acc2_v[bi, pl.ds(base + i * 16, 16)] for i in range(GW))
            accs = lax.fori_loop(0, HR, kbody, acc0)
            for i in range(GW):
                acc2_v[bi, pl.ds(base + i * 16, 16)] = accs[i]
        if half == 1:
            pltpu.sync_copy(acc2_v.at[bi], out_hbm.at[b0 + bi])


@functools.cache
def _sc_gather_kernel():
    return pl.kernel(
        _sc_gather_body,
        out_type=jax.ShapeDtypeStruct((B, D), jnp.float32),
        mesh=plsc.VectorSubcoreMesh(
            core_axis_name="c", subcore_axis_name="s",
            num_cores=2, num_subcores=16),
        scratch_types=[
            pltpu.VMEM((2, K), jnp.int32),
            pltpu.VMEM((2, HR, D), jnp.float32),
            pltpu.VMEM((2, K, 16), jnp.float32),
            pltpu.VMEM((2, D), jnp.float32),
            pltpu.SemaphoreType.DMA,
            pltpu.SemaphoreType.DMA,
        ],
    )


def kernel(feats, mask, seg_quality, W1, b1, W2, b2, Q1, qb1, Q2, qb2):
    maskf = mask.astype(jnp.float32).reshape(B, 1, N)
    weights, attnh, topk_idx, validf, fidx, wk = _score_pool(
        feats, maskf, seg_quality, W1,
        b1.reshape(1, H), W2, b2.reshape(1, 1),
        Q1, qb1.reshape(1, QH), Q2, qb2.reshape(1, 1))
    weights = weights.reshape(B, N)
    attnh = attnh.reshape(B, D)
    feats2d = feats.reshape(B * N, D)
    pooled = _sc_gather_kernel()(feats2d, fidx, wk, attnh)
    return (pooled, weights, topk_idx, validf.astype(bool))
